# baseline (speedup 1.0000x reference)
import functools

import jax
import jax.numpy as jnp
from jax import lax
from jax.experimental import pallas as pl
from jax.experimental.pallas import tpu as pltpu
from jax.experimental.pallas import tpu_sc as plsc

HEIGHT, WIDTH, DIM, BATCH = 32, 32, 768, 64
LANES = 16
NC, NS = 2, 16
NW = NC * NS

_mesh = plsc.VectorSubcoreMesh(core_axis_name="c", subcore_axis_name="s")


@functools.partial(
    pl.kernel,
    mesh=_mesh,
    out_type=jax.ShapeDtypeStruct((BATCH, HEIGHT * WIDTH, DIM), jnp.float32),
    scratch_types=[
        pltpu.VMEM((WIDTH, DIM), jnp.float32),
        pltpu.VMEM((DIM,), jnp.float32),
        pltpu.SemaphoreType.DMA,
    ],
)
def _sc_pos_kernel(row_hbm, col_hbm, out_hbm, buf_v, row_v, sem):
    wid = lax.axis_index("s") * NC + lax.axis_index("c")
    pltpu.sync_copy(col_hbm, buf_v)
    pltpu.sync_copy(row_hbm.at[wid], row_v)

    def add_row(w, carry):
        for j in range(DIM // LANES):
            sl = pl.ds(j * LANES, LANES)
            buf_v[w, sl] = buf_v[w, sl] + row_v[sl]
        return carry

    lax.fori_loop(0, WIDTH, add_row, 0)

    base = wid * WIDTH
    group = 16
    prev = None
    for g in range(BATCH // group):
        cur = [
            pltpu.async_copy(buf_v, out_hbm.at[b, pl.ds(base, WIDTH)], sem)
            for b in range(g * group, (g + 1) * group)
        ]
        if prev is not None:
            for c in prev:
                c.wait()
        prev = cur
    for c in prev:
        c.wait()


def kernel(batch_size, row_embed, col_embed):
    del batch_size
    return _sc_pos_kernel(row_embed, col_embed)
